# baseline (speedup 1.0000x reference)
import jax
import jax.numpy as jnp
import numpy as np
from jax import lax
from jax.experimental import pallas as pl
from jax.experimental.pallas import tpu as pltpu
from jax.experimental.pallas import tpu_sc as plsc

SEQ = 64
IMG = SEQ * SEQ
GRP = 4
NG = SEQ // GRP
GD = GRP * SEQ
BT = 256
IPC = 4

_info = plsc.get_sparse_core_info()
_NC, _NS, _L = _info.num_cores, _info.num_subcores, _info.num_lanes
_NW = _NC * _NS
_CHUNKS = IMG // _L


def _idx_tables():
    c, k = np.meshgrid(np.arange(SEQ), np.arange(SEQ), indexing="ij")
    gather = (k * SEQ + (c - k) % SEQ).reshape(-1).astype(np.int32)
    c, r = np.meshgrid(np.arange(SEQ), np.arange(SEQ), indexing="ij")
    scatter = (r * SEQ + (c - r) % SEQ).reshape(-1).astype(np.int32)
    return gather, scatter


def _gather_body(x_hbm, idx_hbm, z_hbm,
                 idx_v, in0, in1, out0, out1, si0, si1, so0, so1):
    wid = lax.axis_index("s") * _NC + lax.axis_index("c")
    niter = x_hbm.shape[0] // (_NW * IPC)
    base = wid * niter * IPC
    pltpu.sync_copy(idx_hbm, idx_v)
    ins, outs, sis, sos = (in0, in1), (out0, out1), (si0, si1), (so0, so1)

    def issue_in(it, u):
        row = base + it * IPC
        for img in range(IPC):
            pltpu.make_async_copy(
                x_hbm.at[row + img],
                ins[u].at[pl.ds(img * SEQ, SEQ)], sis[u]).start()

    def wait_in(u):
        for img in range(IPC):
            pltpu.make_async_copy(
                x_hbm.at[base],
                ins[u].at[pl.ds(img * SEQ, SEQ)], sis[u]).wait()

    def permute(in_v, out_v):
        @plsc.parallel_loop(0, _CHUNKS, unroll=4)
        def _chunk(q):
            off = q * _L
            f = idx_v[pl.ds(off, _L)]
            fr = lax.shift_right_logical(f, 6)
            fc = f & 63
            for img in range(IPC):
                vals = plsc.load_gather(in_v, [fr + img * SEQ, fc])
                out_v[img, pl.ds(off, _L)] = vals

    issue_in(0, 0)
    issue_in(1, 1)

    def g_body(g, carry):
        for u in (0, 1):
            it = 2 * g + u
            row = base + it * IPC
            wait_in(u)

            @pl.when(it >= 2)
            def _():
                pltpu.make_async_copy(
                    outs[u], z_hbm.at[pl.ds(base, IPC)], sos[u]).wait()

            permute(ins[u], outs[u])
            pltpu.make_async_copy(
                outs[u], z_hbm.at[pl.ds(row, IPC)], sos[u]).start()

            @pl.when(it + 2 < niter)
            def _():
                issue_in(it + 2, u)
        return carry

    lax.fori_loop(0, niter // 2, g_body, 0)
    for u in (0, 1):
        pltpu.make_async_copy(
            outs[u], z_hbm.at[pl.ds(base, IPC)], sos[u]).wait()


def _scatter_body(y_hbm, idx_hbm, o_hbm,
                  idx_v, in0, in1, out0, out1, si0, si1, so0, so1):
    wid = lax.axis_index("s") * _NC + lax.axis_index("c")
    niter = y_hbm.shape[0] // (_NW * IPC)
    base = wid * niter * IPC
    pltpu.sync_copy(idx_hbm, idx_v)
    ins, outs, sis, sos = (in0, in1), (out0, out1), (si0, si1), (so0, so1)

    def issue_out(it, u):
        row = base + it * IPC
        for img in range(IPC):
            pltpu.make_async_copy(
                outs[u].at[pl.ds(img * SEQ, SEQ)],
                o_hbm.at[row + img], sos[u]).start()

    def wait_out(u):
        for img in range(IPC):
            pltpu.make_async_copy(
                outs[u].at[pl.ds(img * SEQ, SEQ)],
                o_hbm.at[base], sos[u]).wait()

    def permute(in_v, out_v):
        @plsc.parallel_loop(0, _CHUNKS, unroll=4)
        def _chunk(q):
            off = q * _L
            f = idx_v[pl.ds(off, _L)]
            fr = lax.shift_right_logical(f, 6)
            fc = f & 63
            for img in range(IPC):
                vals = in_v[img, pl.ds(off, _L)]
                plsc.store_scatter(out_v, [fr + img * SEQ, fc], vals)

    pltpu.make_async_copy(y_hbm.at[pl.ds(base, IPC)], ins[0], sis[0]).start()
    pltpu.make_async_copy(
        y_hbm.at[pl.ds(base + IPC, IPC)], ins[1], sis[1]).start()

    def g_body(g, carry):
        for u in (0, 1):
            it = 2 * g + u
            row = base + it * IPC
            pltpu.make_async_copy(
                y_hbm.at[pl.ds(base, IPC)], ins[u], sis[u]).wait()

            @pl.when(it >= 2)
            def _():
                wait_out(u)

            permute(ins[u], outs[u])
            issue_out(it, u)

            @pl.when(it + 2 < niter)
            def _():
                nrow = base + (it + 2) * IPC
                pltpu.make_async_copy(
                    y_hbm.at[pl.ds(nrow, IPC)], ins[u], sis[u]).start()
        return carry

    lax.fori_loop(0, niter // 2, g_body, 0)
    for u in (0, 1):
        wait_out(u)


def _sc_call(body, arr, idx, out_shape, in_flat):
    mesh = plsc.VectorSubcoreMesh(core_axis_name="c", subcore_axis_name="s")
    in_shape = (IPC * SEQ, SEQ) if not in_flat else (IPC, IMG)
    out_v_shape = (IPC, IMG) if not in_flat else (IPC * SEQ, SEQ)
    fn = pl.kernel(
        body,
        out_type=jax.ShapeDtypeStruct(out_shape, jnp.float32),
        mesh=mesh,
        scratch_types=[
            pltpu.VMEM((IMG,), jnp.int32),
            pltpu.VMEM(in_shape, jnp.float32),
            pltpu.VMEM(in_shape, jnp.float32),
            pltpu.VMEM(out_v_shape, jnp.float32),
            pltpu.VMEM(out_v_shape, jnp.float32),
            pltpu.SemaphoreType.DMA,
            pltpu.SemaphoreType.DMA,
            pltpu.SemaphoreType.DMA,
            pltpu.SemaphoreType.DMA,
        ],
        compiler_params=pltpu.CompilerParams(needs_layout_passes=False),
    )
    return fn(arr, idx)


def _mm_body(z_ref, w_ref, b_ref, o_ref):
    for j in range(NG):
        s = slice(j * GD, (j + 1) * GD)
        o_ref[:, s] = (
            jnp.dot(z_ref[:, s], w_ref[j], preferred_element_type=jnp.float32)
            + b_ref[j][None, :]
        )


def _pack_weights(Ws, bs):
    wt_list, b_list = [], []
    for c in range(SEQ):
        pad = SEQ - 1 - c
        w = jnp.pad(Ws[c], ((0, pad), (0, pad)))
        tail = np.zeros((SEQ,), np.float32)
        tail[c + 1:] = 1.0
        wt_list.append((w + jnp.asarray(np.diag(tail))).T)
        b_list.append(jnp.pad(bs[c], (0, pad)))
    z64 = jnp.zeros((SEQ, SEQ), jnp.float32)
    w4 = jnp.stack([
        jnp.block([[wt_list[GRP * j + t] if t == t2 else z64
                    for t2 in range(GRP)] for t in range(GRP)])
        for j in range(NG)
    ])
    b4 = jnp.stack(b_list).reshape(NG, GD)
    return w4, b4


NCHUNK = 4


def kernel(x, Ws, bs):
    B = x.shape[0]
    w4, b4 = _pack_weights(Ws, bs)
    g_tab, s_tab = _idx_tables()
    g_tab, s_tab = jnp.asarray(g_tab), jnp.asarray(s_tab)

    bc = B // NCHUNK
    mm = pl.pallas_call(
        _mm_body,
        grid=(bc // BT,),
        in_specs=[
            pl.BlockSpec((BT, IMG), lambda i: (i, 0)),
            pl.BlockSpec((NG, GD, GD), lambda i: (0, 0, 0)),
            pl.BlockSpec((NG, GD), lambda i: (0, 0)),
        ],
        out_specs=pl.BlockSpec((BT, IMG), lambda i: (i, 0)),
        out_shape=jax.ShapeDtypeStruct((bc, IMG), jnp.float32),
    )

    outs = []
    for ci in range(NCHUNK):
        xc = lax.slice_in_dim(x, ci * bc, (ci + 1) * bc, axis=0)
        zt = _sc_call(_gather_body, xc, g_tab, (bc, IMG), False)
        y = mm(zt, w4, b4)
        outs.append(_sc_call(_scatter_body, y, s_tab, (bc, SEQ, SEQ), True))
    return jnp.concatenate(outs, axis=0)


# speedup vs baseline: 1.4083x; 1.4083x over previous
import jax
import jax.numpy as jnp
import numpy as np
from jax import lax
from jax.experimental import pallas as pl
from jax.experimental.pallas import tpu as pltpu
from jax.experimental.pallas import tpu_sc as plsc

SEQ = 64
IMG = SEQ * SEQ
GRP = 4
NG = SEQ // GRP
GD = GRP * SEQ
BT = 256
IPC = 4

_info = plsc.get_sparse_core_info()
_NC, _NS, _L = _info.num_cores, _info.num_subcores, _info.num_lanes
_NW = _NC * _NS
_CHUNKS = IMG // _L


def _idx_tables():
    c, k = np.meshgrid(np.arange(SEQ), np.arange(SEQ), indexing="ij")
    gather = (k * SEQ + (c - k) % SEQ).reshape(-1).astype(np.int32)
    c, r = np.meshgrid(np.arange(SEQ), np.arange(SEQ), indexing="ij")
    scatter = (r * SEQ + (c - r) % SEQ).reshape(-1).astype(np.int32)
    return gather, scatter


def _gather_body(x_hbm, idx_hbm, z_hbm,
                 idx_v, in0, in1, out0, out1, si0, si1, so0, so1):
    wid = lax.axis_index("s") * _NC + lax.axis_index("c")
    niter = x_hbm.shape[0] // (_NW * IPC)
    base = wid * niter * IPC
    pltpu.sync_copy(idx_hbm, idx_v)
    ins, outs, sis, sos = (in0, in1), (out0, out1), (si0, si1), (so0, so1)

    def issue_in(it, u):
        row = base + it * IPC
        for img in range(IPC):
            pltpu.make_async_copy(
                x_hbm.at[row + img],
                ins[u].at[pl.ds(img * SEQ, SEQ)], sis[u]).start()

    def wait_in(u):
        for img in range(IPC):
            pltpu.make_async_copy(
                x_hbm.at[base],
                ins[u].at[pl.ds(img * SEQ, SEQ)], sis[u]).wait()

    def permute(in_v, out_v):
        @plsc.parallel_loop(0, _CHUNKS, unroll=4)
        def _chunk(q):
            off = q * _L
            f = idx_v[pl.ds(off, _L)]
            fr = lax.shift_right_logical(f, 6)
            fc = f & 63
            for img in range(IPC):
                vals = plsc.load_gather(in_v, [fr + img * SEQ, fc])
                out_v[img, pl.ds(off, _L)] = vals

    issue_in(0, 0)
    issue_in(1, 1)

    def g_body(g, carry):
        for u in (0, 1):
            it = 2 * g + u
            row = base + it * IPC
            wait_in(u)

            @pl.when(it >= 2)
            def _():
                pltpu.make_async_copy(
                    outs[u], z_hbm.at[pl.ds(base, IPC)], sos[u]).wait()

            permute(ins[u], outs[u])
            pltpu.make_async_copy(
                outs[u], z_hbm.at[pl.ds(row, IPC)], sos[u]).start()

            @pl.when(it + 2 < niter)
            def _():
                issue_in(it + 2, u)
        return carry

    lax.fori_loop(0, niter // 2, g_body, 0)
    for u in (0, 1):
        pltpu.make_async_copy(
            outs[u], z_hbm.at[pl.ds(base, IPC)], sos[u]).wait()


def _scatter_body(y_hbm, idx_hbm, o_hbm,
                  idx_v, in0, in1, out0, out1, si0, si1, so0, so1):
    wid = lax.axis_index("s") * _NC + lax.axis_index("c")
    niter = y_hbm.shape[0] // (_NW * IPC)
    base = wid * niter * IPC
    pltpu.sync_copy(idx_hbm, idx_v)
    ins, outs, sis, sos = (in0, in1), (out0, out1), (si0, si1), (so0, so1)

    def issue_out(it, u):
        row = base + it * IPC
        for img in range(IPC):
            pltpu.make_async_copy(
                outs[u].at[pl.ds(img * SEQ, SEQ)],
                o_hbm.at[row + img], sos[u]).start()

    def wait_out(u):
        for img in range(IPC):
            pltpu.make_async_copy(
                outs[u].at[pl.ds(img * SEQ, SEQ)],
                o_hbm.at[base], sos[u]).wait()

    def permute(in_v, out_v):
        @plsc.parallel_loop(0, _CHUNKS, unroll=4)
        def _chunk(q):
            off = q * _L
            f = idx_v[pl.ds(off, _L)]
            fr = lax.shift_right_logical(f, 6)
            fc = f & 63
            for img in range(IPC):
                vals = in_v[img, pl.ds(off, _L)]
                plsc.store_scatter(out_v, [fr + img * SEQ, fc], vals)

    pltpu.make_async_copy(y_hbm.at[pl.ds(base, IPC)], ins[0], sis[0]).start()
    pltpu.make_async_copy(
        y_hbm.at[pl.ds(base + IPC, IPC)], ins[1], sis[1]).start()

    def g_body(g, carry):
        for u in (0, 1):
            it = 2 * g + u
            row = base + it * IPC
            pltpu.make_async_copy(
                y_hbm.at[pl.ds(base, IPC)], ins[u], sis[u]).wait()

            @pl.when(it >= 2)
            def _():
                wait_out(u)

            permute(ins[u], outs[u])
            issue_out(it, u)

            @pl.when(it + 2 < niter)
            def _():
                nrow = base + (it + 2) * IPC
                pltpu.make_async_copy(
                    y_hbm.at[pl.ds(nrow, IPC)], ins[u], sis[u]).start()
        return carry

    lax.fori_loop(0, niter // 2, g_body, 0)
    for u in (0, 1):
        wait_out(u)


def _sc_call(body, arr, idx, out_shape, in_flat):
    mesh = plsc.VectorSubcoreMesh(core_axis_name="c", subcore_axis_name="s")
    in_shape = (IPC * SEQ, SEQ) if not in_flat else (IPC, IMG)
    out_v_shape = (IPC, IMG) if not in_flat else (IPC * SEQ, SEQ)
    fn = pl.kernel(
        body,
        out_type=jax.ShapeDtypeStruct(out_shape, jnp.float32),
        mesh=mesh,
        scratch_types=[
            pltpu.VMEM((IMG,), jnp.int32),
            pltpu.VMEM(in_shape, jnp.float32),
            pltpu.VMEM(in_shape, jnp.float32),
            pltpu.VMEM(out_v_shape, jnp.float32),
            pltpu.VMEM(out_v_shape, jnp.float32),
            pltpu.SemaphoreType.DMA,
            pltpu.SemaphoreType.DMA,
            pltpu.SemaphoreType.DMA,
            pltpu.SemaphoreType.DMA,
        ],
        compiler_params=pltpu.CompilerParams(
            needs_layout_passes=False, use_tc_tiling_on_sc=True),
    )
    return fn(arr, idx)


def _mm_body(z_ref, w_ref, b_ref, o_ref):
    for j in range(NG):
        s = slice(j * GD, (j + 1) * GD)
        o_ref[:, s] = (
            jnp.dot(z_ref[:, s], w_ref[j], preferred_element_type=jnp.float32)
            + b_ref[j][None, :]
        )


def _pack_weights(Ws, bs):
    wt_list, b_list = [], []
    for c in range(SEQ):
        pad = SEQ - 1 - c
        w = jnp.pad(Ws[c], ((0, pad), (0, pad)))
        tail = np.zeros((SEQ,), np.float32)
        tail[c + 1:] = 1.0
        wt_list.append((w + jnp.asarray(np.diag(tail))).T)
        b_list.append(jnp.pad(bs[c], (0, pad)))
    z64 = jnp.zeros((SEQ, SEQ), jnp.float32)
    w4 = jnp.stack([
        jnp.block([[wt_list[GRP * j + t] if t == t2 else z64
                    for t2 in range(GRP)] for t in range(GRP)])
        for j in range(NG)
    ])
    b4 = jnp.stack(b_list).reshape(NG, GD)
    return w4, b4


def kernel(x, Ws, bs):
    B = x.shape[0]
    w4, b4 = _pack_weights(Ws, bs)
    g_tab, s_tab = _idx_tables()

    zt = _sc_call(_gather_body, x, jnp.asarray(g_tab), (B, IMG), False)

    mm = pl.pallas_call(
        _mm_body,
        grid=(B // BT,),
        in_specs=[
            pl.BlockSpec((BT, IMG), lambda i: (i, 0)),
            pl.BlockSpec((NG, GD, GD), lambda i: (0, 0, 0)),
            pl.BlockSpec((NG, GD), lambda i: (0, 0)),
        ],
        out_specs=pl.BlockSpec((BT, IMG), lambda i: (i, 0)),
        out_shape=jax.ShapeDtypeStruct((B, IMG), jnp.float32),
    )
    y = mm(zt, w4, b4)

    return _sc_call(_scatter_body, y, jnp.asarray(s_tab), (B, SEQ, SEQ), True)
